# baseline (speedup 1.0000x reference)
import functools

import jax
import jax.numpy as jnp
from jax.experimental import pallas as pl
from jax.experimental.pallas import tpu as pltpu


def _fused_gemm_relu(x_ref, w_ref, b_ref, o_ref):
    h = jnp.dot(x_ref[...], w_ref[...], preferred_element_type=jnp.float32)
    o_ref[...] = jnp.maximum(h + b_ref[...], 0.0)


@functools.partial(jax.jit, static_argnames=())
def kernel(x, weights_encode, bias_encode):
    n, d_in = x.shape
    d_out = weights_encode.shape[1]
    block_n = 25000
    grid = (pl.cdiv(n, block_n),)
    bias2d = bias_encode.reshape(1, d_out)
    return pl.pallas_call(
        _fused_gemm_relu,
        grid=grid,
        in_specs=[
            pl.BlockSpec((block_n, d_in), lambda i: (i, 0)),
            pl.BlockSpec((d_in, d_out), lambda i: (0, 0)),
            pl.BlockSpec((1, d_out), lambda i: (0, 0)),
        ],
        out_specs=pl.BlockSpec((block_n, d_out), lambda i: (i, 0)),
        out_shape=jax.ShapeDtypeStruct((n, d_out), jnp.float32),
        compiler_params=pltpu.CompilerParams(
            dimension_semantics=("parallel",),
        ),
    )(x, weights_encode, bias2d)


# speedup vs baseline: 1.0810x; 1.0810x over previous
import functools

import jax
import jax.numpy as jnp
from jax.experimental import pallas as pl
from jax.experimental.pallas import tpu as pltpu


def _fused_gemm_relu(x_ref, w_ref, b_ref, o_ref):
    h = jnp.dot(x_ref[...], w_ref[...], preferred_element_type=jnp.float32)
    o_ref[...] = jnp.maximum(h + b_ref[...], 0.0)


@functools.partial(jax.jit, static_argnames=())
def kernel(x, weights_encode, bias_encode):
    n, d_in = x.shape
    d_out = weights_encode.shape[1]
    block_n = 16000
    grid = (pl.cdiv(n, block_n),)
    bias2d = bias_encode.reshape(1, d_out)
    return pl.pallas_call(
        _fused_gemm_relu,
        grid=grid,
        in_specs=[
            pl.BlockSpec((block_n, d_in), lambda i: (i, 0)),
            pl.BlockSpec((d_in, d_out), lambda i: (0, 0)),
            pl.BlockSpec((1, d_out), lambda i: (0, 0)),
        ],
        out_specs=pl.BlockSpec((block_n, d_out), lambda i: (i, 0)),
        out_shape=jax.ShapeDtypeStruct((n, d_out), jnp.float32),
        compiler_params=pltpu.CompilerParams(
            dimension_semantics=("parallel",),
        ),
    )(x, weights_encode, bias2d)
